# Initial kernel scaffold; baseline (speedup 1.0000x reference)
import jax
import jax.numpy as jnp
from jax.experimental import pallas as pl


def kernel(h, edge_index, edge_weight, edge_attr, batch, W_d2f, b_d2f, W_i, b_i, rW1, rb1, rW2, rb2, W_d, b_d, u, aW1, ab1, aW2, ab2):
    raise NotImplementedError("write your pallas kernel here")


# speedup vs baseline: 2.3876x; 2.3876x over previous
import functools

import jax
import jax.numpy as jnp
from jax import lax
from jax.experimental import pallas as pl
from jax.experimental.pallas import tpu as pltpu, tpu_sc as plsc

N = 10000
E = 320000
F = 128
DF = 16
NR = 5


_GE_BLK = 2000


def _g_body(ea_ref, wt_ref, b_ref, g_ref):
    g_ref[...] = (
        jnp.dot(ea_ref[...], wt_ref[...], preferred_element_type=jnp.float32)
        + b_ref[...]
    )


def _compute_g(edge_attr, W_d2f_t, b_d2f_row):
    grid = (E // _GE_BLK,)
    return pl.pallas_call(
        _g_body,
        grid=grid,
        in_specs=[
            pl.BlockSpec((_GE_BLK, DF), lambda i: (i, 0)),
            pl.BlockSpec((DF, F), lambda i: (0, 0)),
            pl.BlockSpec((1, F), lambda i: (0, 0)),
        ],
        out_specs=pl.BlockSpec((_GE_BLK, F), lambda i: (i, 0)),
        out_shape=jax.ShapeDtypeStruct((E, F), jnp.float32),
    )(edge_attr, W_d2f_t, b_d2f_row)


_NY_BLK = 1000


def _y_body(h_ref, wt_ref, b_ref, y_ref):
    y_ref[...] = (
        jnp.dot(h_ref[...], wt_ref[...], preferred_element_type=jnp.float32)
        + b_ref[...]
    )


def _compute_y(h, W_i_t, b_i_row):
    grid = (N // _NY_BLK,)
    return pl.pallas_call(
        _y_body,
        grid=grid,
        in_specs=[
            pl.BlockSpec((_NY_BLK, F), lambda i: (i, 0)),
            pl.BlockSpec((F, F), lambda i: (0, 0)),
            pl.BlockSpec((1, F), lambda i: (0, 0)),
        ],
        out_specs=pl.BlockSpec((_NY_BLK, F), lambda i: (i, 0)),
        out_shape=jax.ShapeDtypeStruct((N, F), jnp.float32),
    )(h, W_i_t, b_i_row)


_NTILES = 32
_EPT = E // _NTILES
_EC = 80
_NCH = _EPT // _EC
_NSH = 10240
_RPT = _NSH // 16


def _sc_edge_body(y_hbm, src_hbm, dst_hbm, g_hbm, zeros_hbm, out_hbm,
                  m_sh, idx_s, idx_d, yb, gb, sem):
    c = lax.axis_index("c")
    s = lax.axis_index("s")
    t = c * 16 + s

    pltpu.sync_copy(zeros_hbm, m_sh.at[pl.ds(s * _RPT, _RPT)])
    plsc.subcore_barrier()

    def chunk_body(ci, _):
        base = t * _EPT + ci * _EC
        pltpu.sync_copy(src_hbm.at[pl.ds(base, _EC)], idx_s)
        pltpu.sync_copy(dst_hbm.at[pl.ds(base, _EC)], idx_d)
        pltpu.sync_copy(g_hbm.at[pl.ds(base, _EC)], gb)
        pltpu.async_copy(y_hbm.at[idx_s], yb, sem).wait()

        def mul_row(r, carry):
            for j in range(F // 16):
                sl = pl.ds(j * 16, 16)
                gb[r, sl] = gb[r, sl] * yb[r, sl]
            return carry

        lax.fori_loop(0, _EC, mul_row, 0)
        pltpu.sync_copy(gb, m_sh.at[idx_d], add=True)
        return 0

    lax.fori_loop(0, _NCH, chunk_body, 0)
    plsc.subcore_barrier()

    pltpu.sync_copy(m_sh.at[pl.ds(s * _RPT, _RPT)],
                    out_hbm.at[c, pl.ds(s * _RPT, _RPT)])


def _sc_edge(y, src32, dst32, g, zeros_rows):
    mesh = plsc.VectorSubcoreMesh(core_axis_name="c", subcore_axis_name="s")
    kern = functools.partial(
        pl.kernel,
        out_type=jax.ShapeDtypeStruct((2, _NSH, F), jnp.float32),
        mesh=mesh,
        scratch_types=[
            pltpu.VMEM_SHARED((_NSH, F), jnp.float32),
            pltpu.VMEM((_EC,), jnp.int32),
            pltpu.VMEM((_EC,), jnp.int32),
            pltpu.VMEM((_EC, F), jnp.float32),
            pltpu.VMEM((_EC, F), jnp.float32),
            pltpu.SemaphoreType.DMA,
        ],
    )(_sc_edge_body)
    return kern(y, src32, dst32, g, zeros_rows)


_NP_BLK = 1000


def _post_body(m2_ref, h_ref, u_ref, rW1_ref, rb1_ref, rW2_ref, rb2_ref,
               Wd_ref, bd_ref, aW1_ref, ab1_ref, aW2_ref, ab2_ref, x_ref):
    m = m2_ref[0] + m2_ref[1]
    for i in range(NR):
        hmid = jnp.dot(m, rW1_ref[i], preferred_element_type=jnp.float32) + rb1_ref[i]
        m = m + jnp.dot(hmid, rW2_ref[i], preferred_element_type=jnp.float32) + rb2_ref[i]
    x = u_ref[...] * h_ref[...] + (
        jnp.dot(m, Wd_ref[...], preferred_element_type=jnp.float32) + bd_ref[...]
    )
    for i in range(NR):
        hmid = jnp.dot(x, aW1_ref[i], preferred_element_type=jnp.float32) + ab1_ref[i]
        x = x + jnp.dot(hmid, aW2_ref[i], preferred_element_type=jnp.float32) + ab2_ref[i]
    x_ref[...] = x


def _post(m2, h, u_row, rW1t, rb1r, rW2t, rb2r, W_d_t, b_d_row,
          aW1t, ab1r, aW2t, ab2r):
    grid = (N // _NP_BLK,)
    full = lambda shape: pl.BlockSpec(shape, lambda i: tuple(0 for _ in shape))
    return pl.pallas_call(
        _post_body,
        grid=grid,
        in_specs=[
            pl.BlockSpec((2, _NP_BLK, F), lambda i: (0, i, 0)),
            pl.BlockSpec((_NP_BLK, F), lambda i: (i, 0)),
            full((1, F)),
            full((NR, F, F)), full((NR, 1, F)),
            full((NR, F, F)), full((NR, 1, F)),
            full((F, F)), full((1, F)),
            full((NR, F, F)), full((NR, 1, F)),
            full((NR, F, F)), full((NR, 1, F)),
        ],
        out_specs=pl.BlockSpec((_NP_BLK, F), lambda i: (i, 0)),
        out_shape=jax.ShapeDtypeStruct((N, F), jnp.float32),
    )(m2, h, u_row, rW1t, rb1r, rW2t, rb2r, W_d_t, b_d_row,
      aW1t, ab1r, aW2t, ab2r)


def kernel(h, edge_index, edge_weight, edge_attr, batch, W_d2f, b_d2f,
           W_i, b_i, rW1, rb1, rW2, rb2, W_d, b_d, u, aW1, ab1, aW2, ab2):
    del edge_weight, batch

    src32 = edge_index[0].astype(jnp.int32)
    dst32 = edge_index[1].astype(jnp.int32)

    g = _compute_g(edge_attr, W_d2f.T, b_d2f.reshape(1, F))
    y = _compute_y(h, W_i.T, b_i.reshape(1, F))

    zeros_rows = jnp.zeros((_RPT, F), jnp.float32)
    m2 = _sc_edge(y, src32, dst32, g, zeros_rows)

    x = _post(
        m2, h, u.reshape(1, F),
        jnp.swapaxes(rW1, 1, 2), rb1.reshape(NR, 1, F),
        jnp.swapaxes(rW2, 1, 2), rb2.reshape(NR, 1, F),
        W_d.T, b_d.reshape(1, F),
        jnp.swapaxes(aW1, 1, 2), ab1.reshape(NR, 1, F),
        jnp.swapaxes(aW2, 1, 2), ab2.reshape(NR, 1, F),
    )
    return (x, g)


# speedup vs baseline: 3.3776x; 1.4147x over previous
import functools

import jax
import jax.numpy as jnp
from jax import lax
from jax.experimental import pallas as pl
from jax.experimental.pallas import tpu as pltpu, tpu_sc as plsc

N = 10000
E = 320000
F = 128
DF = 16
NR = 5


_GE_BLK = 2000


def _g_body(ea_ref, wt_ref, b_ref, g_ref):
    g_ref[...] = (
        jnp.dot(ea_ref[...], wt_ref[...], preferred_element_type=jnp.float32)
        + b_ref[...]
    )


def _compute_g(edge_attr, W_d2f_t, b_d2f_row):
    grid = (E // _GE_BLK,)
    return pl.pallas_call(
        _g_body,
        grid=grid,
        in_specs=[
            pl.BlockSpec((_GE_BLK, DF), lambda i: (i, 0)),
            pl.BlockSpec((DF, F), lambda i: (0, 0)),
            pl.BlockSpec((1, F), lambda i: (0, 0)),
        ],
        out_specs=pl.BlockSpec((_GE_BLK, F), lambda i: (i, 0)),
        out_shape=jax.ShapeDtypeStruct((E, F), jnp.float32),
    )(edge_attr, W_d2f_t, b_d2f_row)


_NY_BLK = 1000


def _y_body(h_ref, wt_ref, b_ref, y_ref):
    y_ref[...] = (
        jnp.dot(h_ref[...], wt_ref[...], preferred_element_type=jnp.float32)
        + b_ref[...]
    )


def _compute_y(h, W_i_t, b_i_row):
    grid = (N // _NY_BLK,)
    return pl.pallas_call(
        _y_body,
        grid=grid,
        in_specs=[
            pl.BlockSpec((_NY_BLK, F), lambda i: (i, 0)),
            pl.BlockSpec((F, F), lambda i: (0, 0)),
            pl.BlockSpec((1, F), lambda i: (0, 0)),
        ],
        out_specs=pl.BlockSpec((_NY_BLK, F), lambda i: (i, 0)),
        out_shape=jax.ShapeDtypeStruct((N, F), jnp.float32),
    )(h, W_i_t, b_i_row)


_NTILES = 32
_EPT = E // _NTILES
_EC = 80
_NCH = _EPT // _EC
_NSH = 10240
_RPT = _NSH // 16


_LAST = _NCH - 1


def _sc_edge_body(y_hbm, src_hbm, dst_hbm, g_hbm, zeros_hbm, out_hbm,
                  m_sh, gb0, gb1, yb0, yb1,
                  si0, si1, si2, si3, di0, di1,
                  sg0, sg1, sy0, sy1, ss0, ss1,
                  qi0, qi1, qi2, qi3, qd0, qd1):
    c = lax.axis_index("c")
    s = lax.axis_index("s")
    t = c * 16 + s

    gbs, ybs = (gb0, gb1), (yb0, yb1)
    sis, dis = (si0, si1, si2, si3), (di0, di1)
    sgs, sys_, sss = (sg0, sg1), (sy0, sy1), (ss0, ss1)
    qis, qds = (qi0, qi1, qi2, qi3), (qd0, qd1)

    def base(ci):
        return t * _EPT + ci * _EC

    def si_copy(ci, k):
        return pltpu.make_async_copy(src_hbm.at[pl.ds(base(ci), _EC)],
                                     sis[k], qis[k])

    def di_copy(ci, p):
        return pltpu.make_async_copy(dst_hbm.at[pl.ds(base(ci), _EC)],
                                     dis[p], qds[p])

    def g_copy(ci, p):
        return pltpu.make_async_copy(g_hbm.at[pl.ds(base(ci), _EC)],
                                     gbs[p], sgs[p])

    def y_copy(ci, p, k):
        return pltpu.make_async_copy(y_hbm.at[sis[k]], ybs[p], sys_[p])

    def s_copy(p):
        return pltpu.make_async_copy(gbs[p], m_sh.at[dis[p]], sss[p])

    def mul_chunk(p):
        gb, yb = gbs[p], ybs[p]

        @plsc.parallel_loop(0, _EC, unroll=4)
        def _(r):
            for j in range(F // 16):
                sl = pl.ds(j * 16, 16)
                gb[r, sl] = gb[r, sl] * yb[r, sl]

    for k in range(4):
        si_copy(k, k).start()
    for p in range(2):
        di_copy(p, p).start()
        g_copy(p, p).start()
    for p in range(2):
        si_copy(p, p).wait()
        y_copy(p, p, p).start()

    pltpu.sync_copy(zeros_hbm, m_sh.at[pl.ds(s * _RPT, _RPT)])
    plsc.subcore_barrier()

    def chunk_step(ci, k, first=False, do_d=True, do_f=True,
                   guard_d=None, do_h=True, guard_h=None):
        p = k & 1
        g_copy(ci, p).wait()
        y_copy(ci, p, k).wait()
        mul_chunk(p)
        di_copy(ci, p).wait()
        s_copy(p).start(add=True)
        if do_d:
            def start_gather():
                k2 = (k + 2) % 4
                si_copy(ci + 2, k2).wait()
                y_copy(ci + 2, p, k2).start()
            if guard_d is None:
                start_gather()
            else:
                pl.when(guard_d)(start_gather)
        if not first:
            s_copy(1 - p).wait()
            if do_f:
                g_copy(ci + 1, 1 - p).start()
                di_copy(ci + 1, 1 - p).start()
        if do_h:
            def refill():
                si_copy(ci + 4, k).start()
            if guard_h is None:
                refill()
            else:
                pl.when(guard_h)(refill)

    chunk_step(0, 0, first=True)
    for ci in (1, 2, 3):
        chunk_step(ci, ci)

    def quad_body(i4, _):
        a = 4 * i4
        chunk_step(a, 0)
        chunk_step(a + 1, 1, guard_h=(a + 1 <= _LAST - 4))
        chunk_step(a + 2, 2, guard_h=(a + 2 <= _LAST - 4))
        chunk_step(a + 3, 3, guard_d=(a + 3 <= _LAST - 2),
                   guard_h=(a + 3 <= _LAST - 4))
        return 0

    lax.fori_loop(1, 31, quad_body, 0)

    chunk_step(_LAST, 0, do_d=False, do_f=False, do_h=False)
    s_copy(0).wait()
    plsc.subcore_barrier()

    pltpu.sync_copy(m_sh.at[pl.ds(s * _RPT, _RPT)],
                    out_hbm.at[c, pl.ds(s * _RPT, _RPT)])


def _sc_edge(y, src32, dst32, g, zeros_rows):
    mesh = plsc.VectorSubcoreMesh(core_axis_name="c", subcore_axis_name="s")
    kern = functools.partial(
        pl.kernel,
        out_type=jax.ShapeDtypeStruct((2, _NSH, F), jnp.float32),
        mesh=mesh,
        scratch_types=[
            pltpu.VMEM_SHARED((_NSH, F), jnp.float32),
            pltpu.VMEM((_EC, F), jnp.float32),
            pltpu.VMEM((_EC, F), jnp.float32),
            pltpu.VMEM((_EC, F), jnp.float32),
            pltpu.VMEM((_EC, F), jnp.float32),
            pltpu.VMEM((_EC,), jnp.int32),
            pltpu.VMEM((_EC,), jnp.int32),
            pltpu.VMEM((_EC,), jnp.int32),
            pltpu.VMEM((_EC,), jnp.int32),
            pltpu.VMEM((_EC,), jnp.int32),
            pltpu.VMEM((_EC,), jnp.int32),
        ] + [pltpu.SemaphoreType.DMA] * 12,
    )(_sc_edge_body)
    return kern(y, src32, dst32, g, zeros_rows)


_NP_BLK = 1000


def _post_body(m2_ref, h_ref, u_ref, rW1_ref, rb1_ref, rW2_ref, rb2_ref,
               Wd_ref, bd_ref, aW1_ref, ab1_ref, aW2_ref, ab2_ref, x_ref):
    m = m2_ref[0] + m2_ref[1]
    for i in range(NR):
        hmid = jnp.dot(m, rW1_ref[i], preferred_element_type=jnp.float32) + rb1_ref[i]
        m = m + jnp.dot(hmid, rW2_ref[i], preferred_element_type=jnp.float32) + rb2_ref[i]
    x = u_ref[...] * h_ref[...] + (
        jnp.dot(m, Wd_ref[...], preferred_element_type=jnp.float32) + bd_ref[...]
    )
    for i in range(NR):
        hmid = jnp.dot(x, aW1_ref[i], preferred_element_type=jnp.float32) + ab1_ref[i]
        x = x + jnp.dot(hmid, aW2_ref[i], preferred_element_type=jnp.float32) + ab2_ref[i]
    x_ref[...] = x


def _post(m2, h, u_row, rW1t, rb1r, rW2t, rb2r, W_d_t, b_d_row,
          aW1t, ab1r, aW2t, ab2r):
    grid = (N // _NP_BLK,)
    full = lambda shape: pl.BlockSpec(shape, lambda i: tuple(0 for _ in shape))
    return pl.pallas_call(
        _post_body,
        grid=grid,
        in_specs=[
            pl.BlockSpec((2, _NP_BLK, F), lambda i: (0, i, 0)),
            pl.BlockSpec((_NP_BLK, F), lambda i: (i, 0)),
            full((1, F)),
            full((NR, F, F)), full((NR, 1, F)),
            full((NR, F, F)), full((NR, 1, F)),
            full((F, F)), full((1, F)),
            full((NR, F, F)), full((NR, 1, F)),
            full((NR, F, F)), full((NR, 1, F)),
        ],
        out_specs=pl.BlockSpec((_NP_BLK, F), lambda i: (i, 0)),
        out_shape=jax.ShapeDtypeStruct((N, F), jnp.float32),
    )(m2, h, u_row, rW1t, rb1r, rW2t, rb2r, W_d_t, b_d_row,
      aW1t, ab1r, aW2t, ab2r)


def kernel(h, edge_index, edge_weight, edge_attr, batch, W_d2f, b_d2f,
           W_i, b_i, rW1, rb1, rW2, rb2, W_d, b_d, u, aW1, ab1, aW2, ab2):
    del edge_weight, batch

    src32 = edge_index[0].astype(jnp.int32)
    dst32 = edge_index[1].astype(jnp.int32)

    g = _compute_g(edge_attr, W_d2f.T, b_d2f.reshape(1, F))
    y = _compute_y(h, W_i.T, b_i.reshape(1, F))

    zeros_rows = jnp.zeros((_RPT, F), jnp.float32)
    m2 = _sc_edge(y, src32, dst32, g, zeros_rows)

    x = _post(
        m2, h, u.reshape(1, F),
        jnp.swapaxes(rW1, 1, 2), rb1.reshape(NR, 1, F),
        jnp.swapaxes(rW2, 1, 2), rb2.reshape(NR, 1, F),
        W_d.T, b_d.reshape(1, F),
        jnp.swapaxes(aW1, 1, 2), ab1.reshape(NR, 1, F),
        jnp.swapaxes(aW2, 1, 2), ab2.reshape(NR, 1, F),
    )
    return (x, g)


# speedup vs baseline: 3.8133x; 1.1290x over previous
import functools

import jax
import jax.numpy as jnp
from jax import lax
from jax.experimental import pallas as pl
from jax.experimental.pallas import tpu as pltpu, tpu_sc as plsc

N = 10000
E = 320000
F = 128
DF = 16
NR = 5


_GE_BLK = 16000


def _g_body(ea_ref, wt_ref, b_ref, g_ref):
    g_ref[...] = (
        jnp.dot(ea_ref[...], wt_ref[...], preferred_element_type=jnp.float32)
        + b_ref[...]
    )


def _compute_g(edge_attr, W_d2f_t, b_d2f_row):
    grid = (E // _GE_BLK,)
    return pl.pallas_call(
        _g_body,
        grid=grid,
        in_specs=[
            pl.BlockSpec((_GE_BLK, DF), lambda i: (i, 0)),
            pl.BlockSpec((DF, F), lambda i: (0, 0)),
            pl.BlockSpec((1, F), lambda i: (0, 0)),
        ],
        out_specs=pl.BlockSpec((_GE_BLK, F), lambda i: (i, 0)),
        out_shape=jax.ShapeDtypeStruct((E, F), jnp.float32),
    )(edge_attr, W_d2f_t, b_d2f_row)


_NY_BLK = 1000


def _y_body(h_ref, wt_ref, b_ref, y_ref):
    y_ref[...] = (
        jnp.dot(h_ref[...], wt_ref[...], preferred_element_type=jnp.float32)
        + b_ref[...]
    )


def _compute_y(h, W_i_t, b_i_row):
    grid = (N // _NY_BLK,)
    return pl.pallas_call(
        _y_body,
        grid=grid,
        in_specs=[
            pl.BlockSpec((_NY_BLK, F), lambda i: (i, 0)),
            pl.BlockSpec((F, F), lambda i: (0, 0)),
            pl.BlockSpec((1, F), lambda i: (0, 0)),
        ],
        out_specs=pl.BlockSpec((_NY_BLK, F), lambda i: (i, 0)),
        out_shape=jax.ShapeDtypeStruct((N, F), jnp.float32),
    )(h, W_i_t, b_i_row)


_NTILES = 32
_EPT = E // _NTILES
_EC = 80
_NCH = _EPT // _EC
_NSH = 10240
_RPT = _NSH // 16


_LAST = _NCH - 1


def _sc_edge_body(y_hbm, src_hbm, dst_hbm, g_hbm, zeros_hbm, out_hbm,
                  m_sh, gb0, gb1, yb0, yb1,
                  si0, si1, si2, si3, di0, di1,
                  sg0, sg1, sy0, sy1, ss0, ss1,
                  qi0, qi1, qi2, qi3, qd0, qd1):
    c = lax.axis_index("c")
    s = lax.axis_index("s")
    t = c * 16 + s

    gbs, ybs = (gb0, gb1), (yb0, yb1)
    sis, dis = (si0, si1, si2, si3), (di0, di1)
    sgs, sys_, sss = (sg0, sg1), (sy0, sy1), (ss0, ss1)
    qis, qds = (qi0, qi1, qi2, qi3), (qd0, qd1)

    def base(ci):
        return t * _EPT + ci * _EC

    def si_copy(ci, k):
        return pltpu.make_async_copy(src_hbm.at[pl.ds(base(ci), _EC)],
                                     sis[k], qis[k])

    def di_copy(ci, p):
        return pltpu.make_async_copy(dst_hbm.at[pl.ds(base(ci), _EC)],
                                     dis[p], qds[p])

    def g_copy(ci, p):
        return pltpu.make_async_copy(g_hbm.at[pl.ds(base(ci), _EC)],
                                     gbs[p], sgs[p])

    def y_copy(ci, p, k):
        return pltpu.make_async_copy(y_hbm.at[sis[k]], ybs[p], sys_[p])

    def s_copy(p):
        return pltpu.make_async_copy(gbs[p], m_sh.at[dis[p]], sss[p])

    def mul_chunk(p):
        gb, yb = gbs[p], ybs[p]

        @plsc.parallel_loop(0, _EC, unroll=4)
        def _(r):
            for j in range(F // 16):
                sl = pl.ds(j * 16, 16)
                gb[r, sl] = gb[r, sl] * yb[r, sl]

    for k in range(4):
        si_copy(k, k).start()
    for p in range(2):
        di_copy(p, p).start()
        g_copy(p, p).start()
    for p in range(2):
        si_copy(p, p).wait()
        y_copy(p, p, p).start()

    pltpu.sync_copy(zeros_hbm, m_sh.at[pl.ds(s * _RPT, _RPT)])
    plsc.subcore_barrier()

    def chunk_step(ci, k, first=False, do_d=True, do_f=True,
                   guard_d=None, do_h=True, guard_h=None):
        p = k & 1
        g_copy(ci, p).wait()
        y_copy(ci, p, k).wait()
        mul_chunk(p)
        di_copy(ci, p).wait()
        s_copy(p).start(add=True)
        if do_d:
            def start_gather():
                k2 = (k + 2) % 4
                si_copy(ci + 2, k2).wait()
                y_copy(ci + 2, p, k2).start()
            if guard_d is None:
                start_gather()
            else:
                pl.when(guard_d)(start_gather)
        if not first:
            s_copy(1 - p).wait()
            if do_f:
                g_copy(ci + 1, 1 - p).start()
                di_copy(ci + 1, 1 - p).start()
        if do_h:
            def refill():
                si_copy(ci + 4, k).start()
            if guard_h is None:
                refill()
            else:
                pl.when(guard_h)(refill)

    chunk_step(0, 0, first=True)
    for ci in (1, 2, 3):
        chunk_step(ci, ci)

    def quad_body(i4, _):
        a = 4 * i4
        chunk_step(a, 0)
        chunk_step(a + 1, 1, guard_h=(a + 1 <= _LAST - 4))
        chunk_step(a + 2, 2, guard_h=(a + 2 <= _LAST - 4))
        chunk_step(a + 3, 3, guard_d=(a + 3 <= _LAST - 2),
                   guard_h=(a + 3 <= _LAST - 4))
        return 0

    lax.fori_loop(1, 31, quad_body, 0)

    chunk_step(_LAST, 0, do_d=False, do_f=False, do_h=False)
    s_copy(0).wait()
    plsc.subcore_barrier()

    pltpu.sync_copy(m_sh.at[pl.ds(s * _RPT, _RPT)],
                    out_hbm.at[c, pl.ds(s * _RPT, _RPT)])


def _sc_edge(y, src32, dst32, g, zeros_rows):
    mesh = plsc.VectorSubcoreMesh(core_axis_name="c", subcore_axis_name="s")
    kern = functools.partial(
        pl.kernel,
        out_type=jax.ShapeDtypeStruct((2, _NSH, F), jnp.float32),
        mesh=mesh,
        scratch_types=[
            pltpu.VMEM_SHARED((_NSH, F), jnp.float32),
            pltpu.VMEM((_EC, F), jnp.float32),
            pltpu.VMEM((_EC, F), jnp.float32),
            pltpu.VMEM((_EC, F), jnp.float32),
            pltpu.VMEM((_EC, F), jnp.float32),
            pltpu.VMEM((_EC,), jnp.int32),
            pltpu.VMEM((_EC,), jnp.int32),
            pltpu.VMEM((_EC,), jnp.int32),
            pltpu.VMEM((_EC,), jnp.int32),
            pltpu.VMEM((_EC,), jnp.int32),
            pltpu.VMEM((_EC,), jnp.int32),
        ] + [pltpu.SemaphoreType.DMA] * 12,
    )(_sc_edge_body)
    return kern(y, src32, dst32, g, zeros_rows)


_NP_BLK = 1000


def _post_body(m2_ref, h_ref, u_ref, rW1_ref, rb1_ref, rW2_ref, rb2_ref,
               Wd_ref, bd_ref, aW1_ref, ab1_ref, aW2_ref, ab2_ref, x_ref):
    m = m2_ref[0] + m2_ref[1]
    for i in range(NR):
        hmid = jnp.dot(m, rW1_ref[i], preferred_element_type=jnp.float32) + rb1_ref[i]
        m = m + jnp.dot(hmid, rW2_ref[i], preferred_element_type=jnp.float32) + rb2_ref[i]
    x = u_ref[...] * h_ref[...] + (
        jnp.dot(m, Wd_ref[...], preferred_element_type=jnp.float32) + bd_ref[...]
    )
    for i in range(NR):
        hmid = jnp.dot(x, aW1_ref[i], preferred_element_type=jnp.float32) + ab1_ref[i]
        x = x + jnp.dot(hmid, aW2_ref[i], preferred_element_type=jnp.float32) + ab2_ref[i]
    x_ref[...] = x


def _post(m2, h, u_row, rW1t, rb1r, rW2t, rb2r, W_d_t, b_d_row,
          aW1t, ab1r, aW2t, ab2r):
    grid = (N // _NP_BLK,)
    full = lambda shape: pl.BlockSpec(shape, lambda i: tuple(0 for _ in shape))
    return pl.pallas_call(
        _post_body,
        grid=grid,
        in_specs=[
            pl.BlockSpec((2, _NP_BLK, F), lambda i: (0, i, 0)),
            pl.BlockSpec((_NP_BLK, F), lambda i: (i, 0)),
            full((1, F)),
            full((NR, F, F)), full((NR, 1, F)),
            full((NR, F, F)), full((NR, 1, F)),
            full((F, F)), full((1, F)),
            full((NR, F, F)), full((NR, 1, F)),
            full((NR, F, F)), full((NR, 1, F)),
        ],
        out_specs=pl.BlockSpec((_NP_BLK, F), lambda i: (i, 0)),
        out_shape=jax.ShapeDtypeStruct((N, F), jnp.float32),
    )(m2, h, u_row, rW1t, rb1r, rW2t, rb2r, W_d_t, b_d_row,
      aW1t, ab1r, aW2t, ab2r)


def kernel(h, edge_index, edge_weight, edge_attr, batch, W_d2f, b_d2f,
           W_i, b_i, rW1, rb1, rW2, rb2, W_d, b_d, u, aW1, ab1, aW2, ab2):
    del edge_weight, batch

    src32 = edge_index[0].astype(jnp.int32)
    dst32 = edge_index[1].astype(jnp.int32)

    g = _compute_g(edge_attr, W_d2f.T, b_d2f.reshape(1, F))
    y = _compute_y(h, W_i.T, b_i.reshape(1, F))

    zeros_rows = jnp.zeros((_RPT, F), jnp.float32)
    m2 = _sc_edge(y, src32, dst32, g, zeros_rows)

    x = _post(
        m2, h, u.reshape(1, F),
        jnp.swapaxes(rW1, 1, 2), rb1.reshape(NR, 1, F),
        jnp.swapaxes(rW2, 1, 2), rb2.reshape(NR, 1, F),
        W_d.T, b_d.reshape(1, F),
        jnp.swapaxes(aW1, 1, 2), ab1.reshape(NR, 1, F),
        jnp.swapaxes(aW2, 1, 2), ab2.reshape(NR, 1, F),
    )
    return (x, g)
